# baseline (speedup 1.0000x reference)
import functools

import jax
import jax.numpy as jnp
from jax import lax
from jax.experimental import pallas as pl
from jax.experimental.pallas import tpu as pltpu
from jax.experimental.pallas import tpu_sc as plsc

K = 8
DD = 16
D = K * DD
N_ITER = 4
TAU = 1.0

NC = 2
NS = 16
NW = NC * NS
EB = 128
ZR = 128
RB = 512


def _group_matrices():
    row = lax.broadcasted_iota(jnp.int32, (D, K), 0) // DD
    col = lax.broadcasted_iota(jnp.int32, (D, K), 1)
    return (row == col).astype(jnp.float32)


def _caps_normalize(y):
    g = _group_matrices()
    ss = lax.dot_general(y * y, g, (((1,), (0,)), ((), ())),
                         precision=lax.Precision.HIGHEST)
    sse = lax.dot_general(ss, g, (((1,), (1,)), ((), ())),
                          precision=lax.Precision.HIGHEST)
    return y / jnp.maximum(jnp.sqrt(sse), 1e-12)


def _norm_body(x_ref, o_ref):
    o_ref[...] = _caps_normalize(x_ref[...])


def _combine_body(p_ref, x_ref, o_ref):
    y = p_ref[0] + p_ref[1] + x_ref[...]
    o_ref[...] = _caps_normalize(y)


def _tc_normalize(xp):
    nrow = xp.shape[0]
    return pl.pallas_call(
        _norm_body,
        grid=(nrow // RB,),
        in_specs=[pl.BlockSpec((RB, D), lambda i: (i, 0))],
        out_specs=pl.BlockSpec((RB, D), lambda i: (i, 0)),
        out_shape=jax.ShapeDtypeStruct((nrow, D), jnp.float32),
    )(xp)


def _tc_combine(parts, xn):
    nrow = xn.shape[0]
    return pl.pallas_call(
        _combine_body,
        grid=(nrow // RB,),
        in_specs=[
            pl.BlockSpec((NC, RB, D), lambda i: (0, i, 0)),
            pl.BlockSpec((RB, D), lambda i: (i, 0)),
        ],
        out_specs=pl.BlockSpec((RB, D), lambda i: (i, 0)),
        out_shape=jax.ShapeDtypeStruct((nrow, D), jnp.float32),
    )(parts, xn)


def _build_z_body(nb, xn_ref, src_ref, z_ref, sbuf, rbuf):
    c = lax.axis_index("c")
    s = lax.axis_index("s")
    wid = c * NS + s

    def block(b, carry):
        base = (wid * nb + b) * EB
        pltpu.sync_copy(src_ref.at[pl.ds(base, EB)], sbuf)
        pltpu.sync_copy(xn_ref.at[sbuf], rbuf)
        pltpu.sync_copy(rbuf, z_ref.at[pl.ds(base, EB)])
        return carry

    lax.fori_loop(0, nb, block, jnp.int32(0))


def _sc_build_z(xn, src_p, nb, m_pad):
    mesh = plsc.VectorSubcoreMesh(core_axis_name="c", subcore_axis_name="s")
    fn = pl.kernel(
        functools.partial(_build_z_body, nb),
        out_type=jax.ShapeDtypeStruct((m_pad, D), jnp.float32),
        mesh=mesh,
        scratch_types=[
            pltpu.VMEM((EB,), jnp.int32),
            pltpu.VMEM((EB, D), jnp.float32),
        ],
    )
    return fn(xn, src_p)


def _route_body(nb, nrow, z_ref, trg_ref, u_ref, zero_ref, out_ref,
                acc, zbuf, ubuf, tbuf):
    c = lax.axis_index("c")
    s = lax.axis_index("s")
    wid = c * NS + s
    rpt = nrow // NS
    r0 = s * rpt

    for j in range(rpt // ZR):
        pltpu.sync_copy(zero_ref, acc.at[pl.ds(r0 + j * ZR, ZR)])
    plsc.subcore_barrier()

    lanes = lax.iota(jnp.int32, DD)

    def group(g, carry):
        e = g * DD + lanes
        p = []
        for k in range(K):
            accv = jnp.zeros((DD,), jnp.float32)
            for t in range(DD):
                cidx = jnp.full((DD,), k * DD + t, jnp.int32)
                zt = plsc.load_gather(zbuf, [e, cidx])
                ut = plsc.load_gather(ubuf, [e, cidx])
                accv = accv + zt * ut
            p.append(accv)
        mx = p[0]
        for k in range(1, K):
            mx = jnp.maximum(mx, p[k])
        ex = [jnp.exp((p[k] - mx) * (1.0 / TAU)) for k in range(K)]
        tot = ex[0]
        for k in range(1, K):
            tot = tot + ex[k]
        r = [ex[k] / tot for k in range(K)]
        for k in range(K):
            for t in range(DD):
                cidx = jnp.full((DD,), k * DD + t, jnp.int32)
                zt = plsc.load_gather(zbuf, [e, cidx])
                plsc.store_scatter(zbuf, [e, cidx], zt * r[k])
        return carry

    def block(b, carry):
        base = (wid * nb + b) * EB
        pltpu.sync_copy(trg_ref.at[pl.ds(base, EB)], tbuf)
        pltpu.sync_copy(z_ref.at[pl.ds(base, EB)], zbuf)
        pltpu.sync_copy(u_ref.at[tbuf], ubuf)
        lax.fori_loop(0, EB // DD, group, carry)
        pltpu.sync_copy(zbuf, acc.at[tbuf], add=True)
        return carry

    lax.fori_loop(0, nb, block, jnp.int32(0))
    plsc.subcore_barrier()
    pltpu.sync_copy(acc.at[pl.ds(r0, rpt)], out_ref.at[c].at[pl.ds(r0, rpt)])


def _sc_route(z, trg_p, u, zeros_blk, nb, nrow):
    mesh = plsc.VectorSubcoreMesh(core_axis_name="c", subcore_axis_name="s")
    fn = pl.kernel(
        functools.partial(_route_body, nb, nrow),
        out_type=jax.ShapeDtypeStruct((NC, nrow, D), jnp.float32),
        mesh=mesh,
        scratch_types=[
            pltpu.VMEM_SHARED((nrow, D), jnp.float32),
            pltpu.VMEM((EB, D), jnp.float32),
            pltpu.VMEM((EB, D), jnp.float32),
            pltpu.VMEM((EB,), jnp.int32),
        ],
        compiler_params=pltpu.CompilerParams(needs_layout_passes=False),
    )
    return fn(z, trg_p, u, zeros_blk)


def kernel(x, src_trg):
    n, d = x.shape
    assert d == D
    m = src_trg.shape[1]
    st = src_trg.astype(jnp.int32)
    src, trg = st[0], st[1]

    nb = -(-m // (NW * EB))
    m_pad = nb * NW * EB
    nrow = -(-(n + 1) // (NS * ZR)) * (NS * ZR)

    src_p = jnp.pad(src, (0, m_pad - m))
    trg_p = jnp.pad(trg, (0, m_pad - m), constant_values=n)
    x_p = jnp.pad(x, ((0, nrow - n), (0, 0)))
    zeros_blk = jnp.zeros((ZR, D), jnp.float32)

    xn = _tc_normalize(x_p)
    z = _sc_build_z(xn, src_p, nb, m_pad)
    u = xn
    for _ in range(N_ITER):
        parts = _sc_route(z, trg_p, u, zeros_blk, nb, nrow)
        u = _tc_combine(parts, xn)
    return u[:n]
